# Initial kernel scaffold; baseline (speedup 1.0000x reference)
import functools

import jax
import jax.numpy as jnp
from jax import lax
from jax.experimental import pallas as pl
from jax.experimental.pallas import tpu as pltpu
from jax.experimental.pallas import tpu_sc as plsc

N_NODES = 100000
MAX_DEGREE = 64
NUM_SAMPLES = 32
BATCH = 65536

_PERM32 = (35, 45, 31, 63, 7, 4, 29, 44, 16, 58, 37, 19, 61, 2, 34, 5,
           30, 42, 3, 39, 56, 22, 6, 54, 18, 10, 11, 53, 32, 15, 49, 50)

_NC, _NS, _LANES = 2, 16, 16
_NW = _NC * _NS
_BPW = BATCH // _NW
_CHUNK = 128
_NCHUNK = _BPW // _CHUNK


def _body(idx_hbm, table_hbm, perm_hbm, out_hbm, idx_v, rows_v, out_v,
          perm_v, sem):
    wid = lax.axis_index("s") * _NC + lax.axis_index("c")
    base = wid * _BPW
    pltpu.sync_copy(idx_hbm.at[pl.ds(base, _BPW)], idx_v)
    pltpu.sync_copy(perm_hbm, perm_v)
    perm_lo = perm_v[pl.ds(0, _LANES)]
    perm_hi = perm_v[pl.ds(_LANES, _LANES)]

    def chunk_body(c, _):
        pltpu.async_copy(
            table_hbm.at[idx_v.at[pl.ds(c * _CHUNK, _CHUNK)]], rows_v, sem
        ).wait()

        def row_body(r, _):
            rv = jnp.full((_LANES,), r, dtype=jnp.int32)
            g0 = plsc.load_gather(rows_v, [rv, perm_lo])
            g1 = plsc.load_gather(rows_v, [rv, perm_hi])
            out_v[pl.ds(r * NUM_SAMPLES, _LANES)] = g0
            out_v[pl.ds(r * NUM_SAMPLES + _LANES, _LANES)] = g1
            return 0

        lax.fori_loop(0, _CHUNK, row_body, 0)
        pltpu.sync_copy(
            out_v,
            out_hbm.at[pl.ds((base + c * _CHUNK) * NUM_SAMPLES,
                             _CHUNK * NUM_SAMPLES)],
        )
        return 0

    lax.fori_loop(0, _NCHUNK, chunk_body, 0)


@jax.jit
def kernel(inputs, adj_info):
    mesh = plsc.VectorSubcoreMesh(core_axis_name="c", subcore_axis_name="s")
    perm_cols = jnp.asarray(_PERM32, dtype=jnp.int32)
    out_flat = pl.kernel(
        _body,
        out_type=jax.ShapeDtypeStruct((BATCH * NUM_SAMPLES,), jnp.int32),
        mesh=mesh,
        scratch_types=[
            pltpu.VMEM((_BPW,), jnp.int32),
            pltpu.VMEM((_CHUNK, MAX_DEGREE), jnp.int32),
            pltpu.VMEM((_CHUNK * NUM_SAMPLES,), jnp.int32),
            pltpu.VMEM((NUM_SAMPLES,), jnp.int32),
            pltpu.SemaphoreType.DMA,
        ],
    )(inputs, adj_info, perm_cols)
    return out_flat.reshape(BATCH, NUM_SAMPLES)


# speedup vs baseline: 1.0304x; 1.0304x over previous
import functools

import jax
import jax.numpy as jnp
from jax import lax
from jax.experimental import pallas as pl
from jax.experimental.pallas import tpu as pltpu
from jax.experimental.pallas import tpu_sc as plsc

N_NODES = 100000
MAX_DEGREE = 64
NUM_SAMPLES = 32
BATCH = 65536

_PERM32 = (35, 45, 31, 63, 7, 4, 29, 44, 16, 58, 37, 19, 61, 2, 34, 5,
           30, 42, 3, 39, 56, 22, 6, 54, 18, 10, 11, 53, 32, 15, 49, 50)

_NC, _NS, _LANES = 2, 16, 16
_NW = _NC * _NS
_BPW = BATCH // _NW
_CHUNK = 128
_NCHUNK = _BPW // _CHUNK


def _body(idx_hbm, table_hbm, perm_hbm, out_hbm, idx_v, rows_v, out_v,
          perm_v, sem):
    wid = lax.axis_index("s") * _NC + lax.axis_index("c")
    base = wid * _BPW
    pltpu.sync_copy(idx_hbm.at[pl.ds(base, _BPW)], idx_v)
    pltpu.sync_copy(perm_hbm, perm_v)
    perm_lo = perm_v[pl.ds(0, _LANES)]
    perm_hi = perm_v[pl.ds(_LANES, _LANES)]

    def chunk_body(c, _):
        pltpu.async_copy(
            table_hbm.at[idx_v.at[pl.ds(c * _CHUNK, _CHUNK)]], rows_v, sem
        ).wait()

        def row_body(r, _):
            rv = jnp.full((_LANES,), r, dtype=jnp.int32)
            g0 = plsc.load_gather(rows_v, [rv, perm_lo])
            g1 = plsc.load_gather(rows_v, [rv, perm_hi])
            out_v[pl.ds(r * NUM_SAMPLES, _LANES)] = g0
            out_v[pl.ds(r * NUM_SAMPLES + _LANES, _LANES)] = g1
            return 0

        lax.fori_loop(0, _CHUNK, row_body, 0)
        pltpu.sync_copy(
            out_v,
            out_hbm.at[pl.ds((base + c * _CHUNK) * NUM_SAMPLES,
                             _CHUNK * NUM_SAMPLES)],
        )
        return 0

    lax.fori_loop(0, _NCHUNK, chunk_body, 0)


@jax.jit
def kernel(inputs, adj_info):
    mesh = plsc.VectorSubcoreMesh(core_axis_name="c", subcore_axis_name="s")
    perm_cols = jnp.asarray(_PERM32, dtype=jnp.int32)
    out_flat = pl.kernel(
        _body,
        out_type=jax.ShapeDtypeStruct((BATCH * NUM_SAMPLES,), jnp.int32),
        mesh=mesh,
        compiler_params=pltpu.CompilerParams(needs_layout_passes=False,
                                             use_tc_tiling_on_sc=False),
        scratch_types=[
            pltpu.VMEM((_BPW,), jnp.int32),
            pltpu.VMEM((_CHUNK, MAX_DEGREE), jnp.int32),
            pltpu.VMEM((_CHUNK * NUM_SAMPLES,), jnp.int32),
            pltpu.VMEM((NUM_SAMPLES,), jnp.int32),
            pltpu.SemaphoreType.DMA,
        ],
    )(inputs, adj_info, perm_cols)
    return out_flat.reshape(BATCH, NUM_SAMPLES)


# speedup vs baseline: 1.1582x; 1.1240x over previous
import functools

import jax
import jax.numpy as jnp
from jax import lax
from jax.experimental import pallas as pl
from jax.experimental.pallas import tpu as pltpu
from jax.experimental.pallas import tpu_sc as plsc

N_NODES = 100000
MAX_DEGREE = 64
NUM_SAMPLES = 32
BATCH = 65536

_PERM32 = (35, 45, 31, 63, 7, 4, 29, 44, 16, 58, 37, 19, 61, 2, 34, 5,
           30, 42, 3, 39, 56, 22, 6, 54, 18, 10, 11, 53, 32, 15, 49, 50)

_NC, _NS, _LANES = 2, 16, 16
_NW = _NC * _NS
_BPW = BATCH // _NW
_CHUNK = 256
_NCHUNK = _BPW // _CHUNK


def _body(idx_hbm, table_hbm, perm_hbm, out_hbm, idx_v, rows_v, out_v,
          perm_v, gsem0, gsem1, osem0, osem1):
    wid = lax.axis_index("s") * _NC + lax.axis_index("c")
    base = wid * _BPW
    pltpu.sync_copy(idx_hbm.at[pl.ds(base, _BPW)], idx_v)
    pltpu.sync_copy(perm_hbm, perm_v)
    perm_lo = perm_v[pl.ds(0, _LANES)]
    perm_hi = perm_v[pl.ds(_LANES, _LANES)]
    gsems = (gsem0, gsem1)
    osems = (osem0, osem1)

    def fire_gather(g, b):
        pltpu.async_copy(
            table_hbm.at[idx_v.at[pl.ds(g * _CHUNK, _CHUNK)]],
            rows_v.at[b], gsems[b])

    fire_gather(0, 0)
    fire_gather(1, 1)

    def permute_chunk(b):
        rows_b = rows_v.at[b]
        out_b = out_v.at[b]

        def row_body(r, _):
            rv = jnp.full((_LANES,), r, dtype=jnp.int32)
            g0 = plsc.load_gather(rows_b, [rv, perm_lo])
            g1 = plsc.load_gather(rows_b, [rv, perm_hi])
            out_b[pl.ds(r * NUM_SAMPLES, _LANES)] = g0
            out_b[pl.ds(r * NUM_SAMPLES + _LANES, _LANES)] = g1
            return 0

        lax.fori_loop(0, _CHUNK, row_body, 0, unroll=8)

    def super_body(s, _):
        for b in range(2):
            g = s * 2 + b
            pltpu.make_async_copy(
                table_hbm.at[idx_v.at[pl.ds(0, _CHUNK)]],
                rows_v.at[b], gsems[b]).wait()

            @pl.when(g >= 2)
            def _():
                pltpu.make_async_copy(
                    out_v.at[b],
                    out_hbm.at[pl.ds(0, _CHUNK * NUM_SAMPLES)],
                    osems[b]).wait()

            permute_chunk(b)
            pltpu.async_copy(
                out_v.at[b],
                out_hbm.at[pl.ds((base + g * _CHUNK) * NUM_SAMPLES,
                                 _CHUNK * NUM_SAMPLES)],
                osems[b])

            @pl.when(g + 2 < _NCHUNK)
            def _():
                pltpu.async_copy(
                    table_hbm.at[idx_v.at[pl.ds((g + 2) * _CHUNK, _CHUNK)]],
                    rows_v.at[b], gsems[b])
        return 0

    lax.fori_loop(0, _NCHUNK // 2, super_body, 0)
    for b in range(2):
        pltpu.make_async_copy(
            out_v.at[b],
            out_hbm.at[pl.ds(0, _CHUNK * NUM_SAMPLES)],
            osems[b]).wait()


@jax.jit
def kernel(inputs, adj_info):
    mesh = plsc.VectorSubcoreMesh(core_axis_name="c", subcore_axis_name="s")
    perm_cols = jnp.asarray(_PERM32, dtype=jnp.int32)
    out_flat = pl.kernel(
        _body,
        out_type=jax.ShapeDtypeStruct((BATCH * NUM_SAMPLES,), jnp.int32),
        mesh=mesh,
        compiler_params=pltpu.CompilerParams(needs_layout_passes=False,
                                             use_tc_tiling_on_sc=False),
        scratch_types=[
            pltpu.VMEM((_BPW,), jnp.int32),
            pltpu.VMEM((2, _CHUNK, MAX_DEGREE), jnp.int32),
            pltpu.VMEM((2, _CHUNK * NUM_SAMPLES), jnp.int32),
            pltpu.VMEM((NUM_SAMPLES,), jnp.int32),
            pltpu.SemaphoreType.DMA,
            pltpu.SemaphoreType.DMA,
            pltpu.SemaphoreType.DMA,
            pltpu.SemaphoreType.DMA,
        ],
    )(inputs, adj_info, perm_cols)
    return out_flat.reshape(BATCH, NUM_SAMPLES)


# speedup vs baseline: 2.4104x; 2.0812x over previous
import jax
import jax.numpy as jnp
from jax import lax
from jax.experimental import pallas as pl
from jax.experimental.pallas import tpu as pltpu
from jax.experimental.pallas import tpu_sc as plsc

N_NODES = 100000
MAX_DEGREE = 64
NUM_SAMPLES = 32
BATCH = 65536

_PERM32 = (35, 45, 31, 63, 7, 4, 29, 44, 16, 58, 37, 19, 61, 2, 34, 5,
           30, 42, 3, 39, 56, 22, 6, 54, 18, 10, 11, 53, 32, 15, 49, 50)

_NC, _NS, _LANES = 2, 16, 16
_NW = _NC * _NS
_CH = 4096
_NCH = BATCH // _CH


def _body(table_hbm, idx_hbm, out_hbm, col_v, idx_v0, idx_v1, out_v0,
          out_v1, isem0, isem1, osem0, osem1):
    w = lax.axis_index("s") * _NC + lax.axis_index("c")
    p = jnp.int32(0)
    for k in range(_NW):
        p = jnp.where(w == k, jnp.int32(_PERM32[k]), p)
    pltpu.sync_copy(table_hbm.at[pl.ds(p, 1), :], col_v)

    isems = (isem0, isem1)
    osems = (osem0, osem1)
    idx_vs = (idx_v0, idx_v1)
    out_vs = (out_v0, out_v1)
    pltpu.async_copy(idx_hbm.at[pl.ds(0, _CH)], idx_v0, isems[0])
    pltpu.async_copy(idx_hbm.at[pl.ds(_CH, _CH)], idx_v1, isems[1])

    def super_body(s, _):
        for b in range(2):
            g = s * 2 + b
            pltpu.make_async_copy(
                idx_hbm.at[pl.ds(0, _CH)], idx_vs[b], isems[b]).wait()

            @pl.when(g >= 2)
            def _():
                pltpu.make_async_copy(
                    out_vs[b], out_hbm.at[pl.ds(0, 1), pl.ds(0, _CH)],
                    osems[b]).wait()

            idx_b = idx_vs[b]
            out_b = out_vs[b]
            zeros16 = jnp.zeros((_LANES,), dtype=jnp.int32)

            def vec_body(i, _):
                v = idx_b[pl.ds(i * _LANES, _LANES)]
                out_b[0, pl.ds(i * _LANES, _LANES)] = plsc.load_gather(
                    col_v, [zeros16, v])
                return 0

            lax.fori_loop(0, _CH // _LANES, vec_body, 0, unroll=8)
            pltpu.async_copy(
                out_vs[b], out_hbm.at[pl.ds(w, 1), pl.ds(g * _CH, _CH)],
                osems[b])

            @pl.when(g + 2 < _NCH)
            def _():
                pltpu.async_copy(
                    idx_hbm.at[pl.ds((g + 2) * _CH, _CH)], idx_vs[b],
                    isems[b])
        return 0

    lax.fori_loop(0, _NCH // 2, super_body, 0)
    for b in range(2):
        pltpu.make_async_copy(
            out_vs[b], out_hbm.at[pl.ds(0, 1), pl.ds(0, _CH)],
            osems[b]).wait()


@jax.jit
def kernel(inputs, adj_info):
    mesh = plsc.VectorSubcoreMesh(core_axis_name="c", subcore_axis_name="s")
    out_t = pl.kernel(
        _body,
        out_type=jax.ShapeDtypeStruct((NUM_SAMPLES, BATCH), jnp.int32),
        mesh=mesh,
        compiler_params=pltpu.CompilerParams(needs_layout_passes=False),
        scratch_types=[
            pltpu.VMEM((1, N_NODES), jnp.int32),
            pltpu.VMEM((_CH,), jnp.int32),
            pltpu.VMEM((_CH,), jnp.int32),
            pltpu.VMEM((1, _CH), jnp.int32),
            pltpu.VMEM((1, _CH), jnp.int32),
            pltpu.SemaphoreType.DMA,
            pltpu.SemaphoreType.DMA,
            pltpu.SemaphoreType.DMA,
            pltpu.SemaphoreType.DMA,
        ],
    )(adj_info.T, inputs)
    return out_t.T


# speedup vs baseline: 3.4920x; 1.4487x over previous
import jax
import jax.numpy as jnp
from jax import lax
from jax.experimental import pallas as pl
from jax.experimental.pallas import tpu as pltpu
from jax.experimental.pallas import tpu_sc as plsc

N_NODES = 100000
MAX_DEGREE = 64
NUM_SAMPLES = 32
BATCH = 65536

_PERM32 = (35, 45, 31, 63, 7, 4, 29, 44, 16, 58, 37, 19, 61, 2, 34, 5,
           30, 42, 3, 39, 56, 22, 6, 54, 18, 10, 11, 53, 32, 15, 49, 50)

_NC, _NS, _LANES = 2, 16, 16
_NW = _NC * _NS
_CH = 4096
_NCH = BATCH // _CH


def _body(table_hbm, idx_hbm, out_hbm, col_v, idx_v0, idx_v1, out_v0,
          out_v1, isem0, isem1, osem0, osem1):
    w = lax.axis_index("s") * _NC + lax.axis_index("c")
    p = jnp.int32(0)
    for k in range(_NW):
        p = jnp.where(w == k, jnp.int32(_PERM32[k]), p)
    pltpu.sync_copy(table_hbm.at[pl.ds(p, 1), :], col_v)

    isems = (isem0, isem1)
    osems = (osem0, osem1)
    idx_vs = (idx_v0, idx_v1)
    out_vs = (out_v0, out_v1)
    pltpu.async_copy(idx_hbm.at[pl.ds(0, _CH)], idx_v0, isems[0])
    pltpu.async_copy(idx_hbm.at[pl.ds(_CH, _CH)], idx_v1, isems[1])

    def super_body(s, _):
        for b in range(2):
            g = s * 2 + b
            pltpu.make_async_copy(
                idx_hbm.at[pl.ds(0, _CH)], idx_vs[b], isems[b]).wait()

            @pl.when(g >= 2)
            def _():
                pltpu.make_async_copy(
                    out_vs[b], out_hbm.at[pl.ds(0, 1), pl.ds(0, _CH)],
                    osems[b]).wait()

            idx_b = idx_vs[b]
            out_b = out_vs[b]
            zeros16 = jnp.zeros((_LANES,), dtype=jnp.int32)

            @plsc.parallel_loop(0, _CH, _LANES, unroll=8)
            def _(i):
                v = idx_b[pl.ds(i, _LANES)]
                out_b[0, pl.ds(i, _LANES)] = plsc.load_gather(
                    col_v, [zeros16, v])
            pltpu.async_copy(
                out_vs[b], out_hbm.at[pl.ds(w, 1), pl.ds(g * _CH, _CH)],
                osems[b])

            @pl.when(g + 2 < _NCH)
            def _():
                pltpu.async_copy(
                    idx_hbm.at[pl.ds((g + 2) * _CH, _CH)], idx_vs[b],
                    isems[b])
        return 0

    lax.fori_loop(0, _NCH // 2, super_body, 0)
    for b in range(2):
        pltpu.make_async_copy(
            out_vs[b], out_hbm.at[pl.ds(0, 1), pl.ds(0, _CH)],
            osems[b]).wait()


@jax.jit
def kernel(inputs, adj_info):
    mesh = plsc.VectorSubcoreMesh(core_axis_name="c", subcore_axis_name="s")
    out_t = pl.kernel(
        _body,
        out_type=jax.ShapeDtypeStruct((NUM_SAMPLES, BATCH), jnp.int32),
        mesh=mesh,
        compiler_params=pltpu.CompilerParams(needs_layout_passes=False),
        scratch_types=[
            pltpu.VMEM((1, N_NODES), jnp.int32),
            pltpu.VMEM((_CH,), jnp.int32),
            pltpu.VMEM((_CH,), jnp.int32),
            pltpu.VMEM((1, _CH), jnp.int32),
            pltpu.VMEM((1, _CH), jnp.int32),
            pltpu.SemaphoreType.DMA,
            pltpu.SemaphoreType.DMA,
            pltpu.SemaphoreType.DMA,
            pltpu.SemaphoreType.DMA,
        ],
    )(adj_info.T, inputs)
    return out_t.T
